# baseline (speedup 1.0000x reference)
import functools
import jax
import jax.numpy as jnp
from jax import lax
from jax.experimental import pallas as pl
from jax.experimental.pallas import tpu as pltpu
from jax.experimental.pallas import tpu_sc as plsc

_NC = 2
_NS = 16
_NW = _NC * _NS
_C = 80
_NBUF = 4


def _gather_call(x, idx):
    E = idx.shape[0]
    N = x.shape[0]
    D = x.shape[1]
    b_per_w = E // _NW
    n_per_s = N // _NS
    C = _C
    nbuf = _NBUF
    n_chunks = b_per_w // C
    n_groups = -(-n_chunks // nbuf)

    mesh = plsc.VectorSubcoreMesh(core_axis_name="c", subcore_axis_name="s")

    @functools.partial(
        pl.kernel,
        out_type=jax.ShapeDtypeStruct((E, D), jnp.float32),
        mesh=mesh,
        scratch_types=(
            [pltpu.VMEM_SHARED((N, D), jnp.float32),
             pltpu.VMEM((b_per_w,), jnp.int32)]
            + [pltpu.VMEM((C, D), jnp.float32) for _ in range(nbuf)]
            + [pltpu.SemaphoreType.DMA for _ in range(2 * nbuf)]
        ),
    )
    def gather_kernel(x_hbm, idx_hbm, out_hbm, table_s, idx_v, *bufs):
        rows_b = bufs[:nbuf]
        gsem = bufs[nbuf:2 * nbuf]
        wsem = bufs[2 * nbuf:]
        sid = lax.axis_index("s")
        wid = sid * _NC + lax.axis_index("c")
        wbase = wid * b_per_w

        n_al = (N // _NS) // 8 * 8
        tail = N - _NS * n_al
        lo_t = pl.multiple_of(sid * n_al, 8)
        pltpu.sync_copy(x_hbm.at[pl.ds(lo_t, n_al)],
                        table_s.at[pl.ds(lo_t, n_al)])

        @pl.when(sid == 0)
        def _stage_tail():
            lo = pl.multiple_of(_NS * n_al, 8)
            pltpu.sync_copy(x_hbm.at[pl.ds(lo, tail)],
                            table_s.at[pl.ds(lo, tail)])
        pltpu.sync_copy(idx_hbm.at[pl.ds(pl.multiple_of(wbase, 8), b_per_w)],
                        idx_v)
        plsc.subcore_barrier()

        def issue_gather(c, b):
            lo = pl.multiple_of(c * C, 8)
            pltpu.async_copy(
                table_s.at[idx_v.at[pl.ds(lo, C)]], rows_b[b], gsem[b])

        def wait_gather(c, b):
            lo = pl.multiple_of(c * C, 8)
            pltpu.make_async_copy(
                table_s.at[idx_v.at[pl.ds(lo, C)]], rows_b[b], gsem[b]).wait()

        def issue_wb(c, b):
            lo = pl.multiple_of(wbase + c * C, 8)
            pltpu.async_copy(rows_b[b], out_hbm.at[pl.ds(lo, C)], wsem[b])

        def wait_wb(c, b):
            lo = pl.multiple_of(wbase + c * C, 8)
            pltpu.make_async_copy(
                rows_b[b], out_hbm.at[pl.ds(lo, C)], wsem[b]).wait()

        for b in range(nbuf):
            if b < n_chunks:
                issue_gather(b, b)

        def body(j, carry):
            for b in range(nbuf):
                c = j * nbuf + b

                @pl.when(c < n_chunks)
                def _fire():
                    wait_gather(c, b)
                    issue_wb(c, b)

            for b in range(nbuf):
                c = j * nbuf + b

                @pl.when(c < n_chunks)
                def _drain():
                    wait_wb(c, b)

                    @pl.when(c + nbuf < n_chunks)
                    def _next():
                        issue_gather(c + nbuf, b)
            return carry

        lax.fori_loop(0, n_groups, body, 0)

    return gather_kernel(x, idx)


def kernel(x, species, edge_src, edge_dst):
    return _gather_call(x, edge_dst)


# speedup vs baseline: 1.0141x; 1.0141x over previous
import functools
import jax
import jax.numpy as jnp
from jax import lax
from jax.experimental import pallas as pl
from jax.experimental.pallas import tpu as pltpu
from jax.experimental.pallas import tpu_sc as plsc

_NC = 2
_NS = 16
_NW = _NC * _NS
_C = 80
_NBUF = 4


def _gather_call(x, idx):
    E = idx.shape[0]
    N = x.shape[0]
    D = x.shape[1]
    b_per_w = E // _NW
    n_per_s = N // _NS
    C = _C
    nbuf = _NBUF
    n_chunks = b_per_w // C
    n_groups = -(-n_chunks // nbuf)

    mesh = plsc.VectorSubcoreMesh(core_axis_name="c", subcore_axis_name="s")

    @functools.partial(
        pl.kernel,
        out_type=jax.ShapeDtypeStruct((E, D), jnp.float32),
        mesh=mesh,
        scratch_types=(
            [pltpu.VMEM_SHARED((N, D), jnp.float32),
             pltpu.VMEM((b_per_w,), jnp.int32)]
            + [pltpu.VMEM((C, D), jnp.float32) for _ in range(nbuf)]
            + [pltpu.SemaphoreType.DMA for _ in range(2 * nbuf)]
        ),
    )
    def gather_kernel(x_hbm, idx_hbm, out_hbm, table_s, idx_v, *bufs):
        rows_b = bufs[:nbuf]
        gsem = bufs[nbuf:2 * nbuf]
        wsem = bufs[2 * nbuf:]
        sid = lax.axis_index("s")
        wid = sid * _NC + lax.axis_index("c")
        wbase = wid * b_per_w

        n_al = (N // _NS) // 8 * 8
        tail = N - _NS * n_al
        lo_t = pl.multiple_of(sid * n_al, 8)
        pltpu.sync_copy(x_hbm.at[pl.ds(lo_t, n_al)],
                        table_s.at[pl.ds(lo_t, n_al)])

        @pl.when(sid == 0)
        def _stage_tail():
            lo = pl.multiple_of(_NS * n_al, 8)
            pltpu.sync_copy(x_hbm.at[pl.ds(lo, tail)],
                            table_s.at[pl.ds(lo, tail)])
        pltpu.sync_copy(idx_hbm.at[pl.ds(pl.multiple_of(wbase, 8), b_per_w)],
                        idx_v)
        plsc.subcore_barrier()

        def issue_gather(c, b):
            lo = pl.multiple_of(c * C, 8)
            pltpu.async_copy(
                table_s.at[idx_v.at[pl.ds(lo, C)]], rows_b[b], gsem[b])

        def wait_gather(c, b):
            lo = pl.multiple_of(c * C, 8)
            pltpu.make_async_copy(
                table_s.at[idx_v.at[pl.ds(lo, C)]], rows_b[b], gsem[b]).wait()

        def issue_wb(c, b):
            lo = pl.multiple_of(wbase + c * C, 8)
            pltpu.async_copy(rows_b[b], out_hbm.at[pl.ds(lo, C)], wsem[b])

        def wait_wb(c, b):
            lo = pl.multiple_of(wbase + c * C, 8)
            pltpu.make_async_copy(
                rows_b[b], out_hbm.at[pl.ds(lo, C)], wsem[b]).wait()

        for b in range(nbuf):
            if b < n_chunks:
                issue_gather(b, b)

        skew = 2

        def body(j, carry):
            for b in range(nbuf):
                c = j * nbuf + b

                @pl.when(c < n_chunks)
                def _turn():
                    wait_gather(c, b)
                    issue_wb(c, b)

                cd = c - skew
                bd = (b + skew) % nbuf

                @pl.when((cd >= 0) & (cd < n_chunks))
                def _drain():
                    wait_wb(cd, bd)

                    @pl.when(cd + nbuf < n_chunks)
                    def _next():
                        issue_gather(cd + nbuf, bd)
            return carry

        lax.fori_loop(0, -(-(n_chunks + skew) // nbuf), body, 0)

    return gather_kernel(x, idx)


def kernel(x, species, edge_src, edge_dst):
    return _gather_call(x, edge_dst)


# speedup vs baseline: 1.0482x; 1.0336x over previous
import functools
import jax
import jax.numpy as jnp
from jax import lax
from jax.experimental import pallas as pl
from jax.experimental.pallas import tpu as pltpu
from jax.experimental.pallas import tpu_sc as plsc

_NC = 2
_NS = 16
_NW = _NC * _NS
_C = 80
_NBUF = 4


def _gather_call(x, idx):
    E = idx.shape[0]
    N = x.shape[0]
    D = x.shape[1]
    b_per_w = E // _NW
    n_per_s = N // _NS
    C = _C
    nbuf = _NBUF
    n_chunks = b_per_w // C
    n_groups = -(-n_chunks // nbuf)

    mesh = plsc.VectorSubcoreMesh(core_axis_name="c", subcore_axis_name="s")

    @functools.partial(
        pl.kernel,
        out_type=jax.ShapeDtypeStruct((E, D), jnp.float32),
        mesh=mesh,
        scratch_types=(
            [pltpu.VMEM_SHARED((N, D), jnp.float32),
             pltpu.VMEM((b_per_w,), jnp.int32)]
            + [pltpu.VMEM((C, D), jnp.float32) for _ in range(nbuf)]
            + [pltpu.SemaphoreType.DMA for _ in range(2 * nbuf + 2)]
        ),
    )
    def gather_kernel(x_hbm, idx_hbm, out_hbm, table_s, idx_v, *bufs):
        rows_b = bufs[:nbuf]
        gsem = bufs[nbuf:2 * nbuf]
        wsem = bufs[2 * nbuf:3 * nbuf]
        t_sem, i_sem = bufs[3 * nbuf:]
        sid = lax.axis_index("s")
        wid = sid * _NC + lax.axis_index("c")
        wbase = wid * b_per_w

        n_al = (N // _NS) // 8 * 8
        tail = N - _NS * n_al
        lo_t = pl.multiple_of(sid * n_al, 8)
        lo_tail = pl.multiple_of(_NS * n_al, 8)
        idx_lo = pl.multiple_of(wbase, 8)
        pltpu.async_copy(x_hbm.at[pl.ds(lo_t, n_al)],
                         table_s.at[pl.ds(lo_t, n_al)], t_sem)
        pltpu.async_copy(idx_hbm.at[pl.ds(idx_lo, b_per_w)], idx_v, i_sem)

        @pl.when(sid == 0)
        def _stage_tail():
            pltpu.async_copy(x_hbm.at[pl.ds(lo_tail, tail)],
                             table_s.at[pl.ds(lo_tail, tail)], t_sem)

        pltpu.make_async_copy(x_hbm.at[pl.ds(lo_t, n_al)],
                              table_s.at[pl.ds(lo_t, n_al)], t_sem).wait()
        pltpu.make_async_copy(idx_hbm.at[pl.ds(idx_lo, b_per_w)],
                              idx_v, i_sem).wait()

        @pl.when(sid == 0)
        def _wait_tail():
            pltpu.make_async_copy(x_hbm.at[pl.ds(lo_tail, tail)],
                                  table_s.at[pl.ds(lo_tail, tail)],
                                  t_sem).wait()

        plsc.subcore_barrier()

        def issue_gather(c, b):
            lo = pl.multiple_of(c * C, 8)
            pltpu.async_copy(
                table_s.at[idx_v.at[pl.ds(lo, C)]], rows_b[b], gsem[b])

        def wait_gather(c, b):
            lo = pl.multiple_of(c * C, 8)
            pltpu.make_async_copy(
                table_s.at[idx_v.at[pl.ds(lo, C)]], rows_b[b], gsem[b]).wait()

        def issue_wb(c, b):
            lo = pl.multiple_of(wbase + c * C, 8)
            pltpu.async_copy(rows_b[b], out_hbm.at[pl.ds(lo, C)], wsem[b])

        def wait_wb(c, b):
            lo = pl.multiple_of(wbase + c * C, 8)
            pltpu.make_async_copy(
                rows_b[b], out_hbm.at[pl.ds(lo, C)], wsem[b]).wait()

        for b in range(nbuf):
            if b < n_chunks:
                issue_gather(b, b)

        def body(j, carry):
            for b in range(nbuf):
                c = j * nbuf + b

                @pl.when(c < n_chunks)
                def _turn():
                    wait_gather(c, b)
                    issue_wb(c, b)
                    wait_wb(c, b)

                    @pl.when(c + nbuf < n_chunks)
                    def _next():
                        issue_gather(c + nbuf, b)
            return carry

        lax.fori_loop(0, n_groups, body, 0)

    return gather_kernel(x, idx)


def kernel(x, species, edge_src, edge_dst):
    return _gather_call(x, edge_dst)
